# Initial kernel scaffold; baseline (speedup 1.0000x reference)
import jax
import jax.numpy as jnp
from jax.experimental import pallas as pl
from jax.experimental.pallas import tpu as pltpu

_T = 8


def _scan_body(a_ref, inT_ref, bst_ref, o_ref, x_ref):
    k = pl.program_id(0)

    @pl.when(k == 0)
    def _():
        x_ref[...] = jnp.zeros_like(x_ref)

    x = x_ref[...]
    for j in range(_T):
        u = inT_ref[:, j : j + 1] * bst_ref[j : j + 1, :]
        x = u + jax.lax.dot_general(
            x, a_ref[j], (((1,), (1,)), ((), ())),
            preferred_element_type=jnp.float32,
        )
        o_ref[j] = x
    x_ref[...] = x


def kernel(inputs, A_stacked, B_stacked):
    L, B = inputs.shape
    N = A_stacked.shape[-1]
    inputs_t = inputs.T

    grid = (L // _T,)
    return pl.pallas_call(
        _scan_body,
        out_shape=jax.ShapeDtypeStruct((L, B, N), jnp.float32),
        grid=grid,
        in_specs=[
            pl.BlockSpec((_T, N, N), lambda k: (k, 0, 0)),
            pl.BlockSpec((B, _T), lambda k: (0, k)),
            pl.BlockSpec((_T, N), lambda k: (k, 0)),
        ],
        out_specs=pl.BlockSpec((_T, B, N), lambda k: (k, 0, 0)),
        scratch_shapes=[pltpu.VMEM((B, N), jnp.float32)],
        compiler_params=pltpu.CompilerParams(
            dimension_semantics=("arbitrary",),
        ),
        name="hippo_legs_scan",
    )(A_stacked, inputs_t, B_stacked)


# speedup vs baseline: 9.7227x; 9.7227x over previous
import jax
import jax.numpy as jnp
from jax.experimental import pallas as pl
from jax.experimental.pallas import tpu as pltpu

_T = 8


def _scan_body(a_ref, inT_ref, bst_ref, o_ref, x_ref):
    k = pl.program_id(0)

    @pl.when(k == 0)
    def _():
        x_ref[...] = jnp.zeros_like(x_ref)

    x = x_ref[...]
    for j in range(_T):
        u = inT_ref[0, :, j : j + 1] * bst_ref[j : j + 1, :]
        x = u + jax.lax.dot_general(
            x, a_ref[j], (((1,), (1,)), ((), ())),
            preferred_element_type=jnp.float32,
        )
        o_ref[j] = x
    x_ref[...] = x


def kernel(inputs, A_stacked, B_stacked):
    L, B = inputs.shape
    N = A_stacked.shape[-1]
    inputs_t = inputs.reshape(L // _T, _T, B).transpose(0, 2, 1)

    grid = (L // _T,)
    return pl.pallas_call(
        _scan_body,
        out_shape=jax.ShapeDtypeStruct((L, B, N), jnp.float32),
        grid=grid,
        in_specs=[
            pl.BlockSpec((_T, N, N), lambda k: (k, 0, 0)),
            pl.BlockSpec((1, B, _T), lambda k: (k, 0, 0)),
            pl.BlockSpec((_T, N), lambda k: (k, 0)),
        ],
        out_specs=pl.BlockSpec((_T, B, N), lambda k: (k, 0, 0)),
        scratch_shapes=[pltpu.VMEM((B, N), jnp.float32)],
        compiler_params=pltpu.CompilerParams(
            dimension_semantics=("arbitrary",),
        ),
        name="hippo_legs_scan",
    )(A_stacked, inputs_t, B_stacked)


# speedup vs baseline: 11.8001x; 1.2137x over previous
import jax
import jax.numpy as jnp
from jax.experimental import pallas as pl
from jax.experimental.pallas import tpu as pltpu

_T = 16


def _scan_body(a_ref, inT_ref, bst_ref, o_ref, x_ref):
    k = pl.program_id(0)

    @pl.when(k == 0)
    def _():
        x_ref[...] = jnp.zeros_like(x_ref)

    x = x_ref[...]
    for j in range(_T):
        u = inT_ref[0, :, j : j + 1] * bst_ref[j : j + 1, :]
        x = u + jax.lax.dot_general(
            x, a_ref[j], (((1,), (1,)), ((), ())),
            preferred_element_type=jnp.float32,
        )
        o_ref[j] = x
    x_ref[...] = x


def kernel(inputs, A_stacked, B_stacked):
    L, B = inputs.shape
    N = A_stacked.shape[-1]
    inputs_t = inputs.reshape(L // _T, _T, B).transpose(0, 2, 1)

    grid = (L // _T,)
    return pl.pallas_call(
        _scan_body,
        out_shape=jax.ShapeDtypeStruct((L, B, N), jnp.float32),
        grid=grid,
        in_specs=[
            pl.BlockSpec((_T, N, N), lambda k: (k, 0, 0)),
            pl.BlockSpec((1, B, _T), lambda k: (k, 0, 0)),
            pl.BlockSpec((_T, N), lambda k: (k, 0)),
        ],
        out_specs=pl.BlockSpec((_T, B, N), lambda k: (k, 0, 0)),
        scratch_shapes=[pltpu.VMEM((B, N), jnp.float32)],
        compiler_params=pltpu.CompilerParams(
            dimension_semantics=("arbitrary",),
        ),
        name="hippo_legs_scan",
    )(A_stacked, inputs_t, B_stacked)


# speedup vs baseline: 13.1452x; 1.1140x over previous
import jax
import jax.numpy as jnp
from jax.experimental import pallas as pl
from jax.experimental.pallas import tpu as pltpu

_T = 32


def _scan_body(a_ref, inT_ref, bst_ref, o_ref, x_ref):
    k = pl.program_id(0)

    @pl.when(k == 0)
    def _():
        x_ref[...] = jnp.zeros_like(x_ref)

    x = x_ref[...]
    for j in range(_T):
        u = inT_ref[0, :, j : j + 1] * bst_ref[j : j + 1, :]
        x = u + jax.lax.dot_general(
            x, a_ref[j], (((1,), (1,)), ((), ())),
            preferred_element_type=jnp.float32,
        )
        o_ref[j] = x
    x_ref[...] = x


def kernel(inputs, A_stacked, B_stacked):
    L, B = inputs.shape
    N = A_stacked.shape[-1]
    inputs_t = inputs.reshape(L // _T, _T, B).transpose(0, 2, 1)

    grid = (L // _T,)
    return pl.pallas_call(
        _scan_body,
        out_shape=jax.ShapeDtypeStruct((L, B, N), jnp.float32),
        grid=grid,
        in_specs=[
            pl.BlockSpec((_T, N, N), lambda k: (k, 0, 0)),
            pl.BlockSpec((1, B, _T), lambda k: (k, 0, 0)),
            pl.BlockSpec((_T, N), lambda k: (k, 0)),
        ],
        out_specs=pl.BlockSpec((_T, B, N), lambda k: (k, 0, 0)),
        scratch_shapes=[pltpu.VMEM((B, N), jnp.float32)],
        compiler_params=pltpu.CompilerParams(
            dimension_semantics=("arbitrary",),
            vmem_limit_bytes=56 * 1024 * 1024,
        ),
        name="hippo_legs_scan",
    )(A_stacked, inputs_t, B_stacked)


# speedup vs baseline: 14.3794x; 1.0939x over previous
import jax
import jax.numpy as jnp
from jax.experimental import pallas as pl
from jax.experimental.pallas import tpu as pltpu

_T = 32


def _scan_body(a11_ref, abot_ref, inT_ref, bst_ref, o_ref, x_ref):
    k = pl.program_id(0)

    @pl.when(k == 0)
    def _():
        x_ref[...] = jnp.zeros_like(x_ref)

    h = a11_ref.shape[1]
    zeros = jnp.zeros((h, h), jnp.float32)
    x = x_ref[...]
    for j in range(_T):
        u = inT_ref[0, :, j : j + 1] * bst_ref[j : j + 1, :]
        top = jnp.concatenate([a11_ref[j], zeros], axis=1)
        a_full = jnp.concatenate([top, abot_ref[j]], axis=0)
        x = u + jax.lax.dot_general(
            x, a_full, (((1,), (1,)), ((), ())),
            preferred_element_type=jnp.float32,
        )
        o_ref[j] = x
    x_ref[...] = x


def kernel(inputs, A_stacked, B_stacked):
    L, B = inputs.shape
    N = A_stacked.shape[-1]
    h = N // 2
    inputs_t = inputs.reshape(L // _T, _T, B).transpose(0, 2, 1)

    grid = (L // _T,)
    return pl.pallas_call(
        _scan_body,
        out_shape=jax.ShapeDtypeStruct((L, B, N), jnp.float32),
        grid=grid,
        in_specs=[
            pl.BlockSpec((_T, h, h), lambda k: (k, 0, 0)),
            pl.BlockSpec((_T, h, N), lambda k: (k, 1, 0)),
            pl.BlockSpec((1, B, _T), lambda k: (k, 0, 0)),
            pl.BlockSpec((_T, N), lambda k: (k, 0)),
        ],
        out_specs=pl.BlockSpec((_T, B, N), lambda k: (k, 0, 0)),
        scratch_shapes=[pltpu.VMEM((B, N), jnp.float32)],
        compiler_params=pltpu.CompilerParams(
            dimension_semantics=("arbitrary",),
            vmem_limit_bytes=56 * 1024 * 1024,
        ),
        name="hippo_legs_scan",
    )(A_stacked, A_stacked, inputs_t, B_stacked)


# speedup vs baseline: 14.9111x; 1.0370x over previous
import jax
import jax.numpy as jnp
from jax.experimental import pallas as pl
from jax.experimental.pallas import tpu as pltpu

_T = 64


def _scan_body(a11_ref, abot_ref, inT_ref, bst_ref, o_ref, x_ref):
    k = pl.program_id(0)

    @pl.when(k == 0)
    def _():
        x_ref[...] = jnp.zeros_like(x_ref)

    h = a11_ref.shape[1]
    zeros = jnp.zeros((h, h), jnp.float32)
    x = x_ref[...]
    for j in range(_T):
        u = inT_ref[0, :, j : j + 1] * bst_ref[j : j + 1, :]
        top = jnp.concatenate([a11_ref[j], zeros], axis=1)
        a_full = jnp.concatenate([top, abot_ref[j]], axis=0)
        x = u + jax.lax.dot_general(
            x, a_full, (((1,), (1,)), ((), ())),
            preferred_element_type=jnp.float32,
        )
        o_ref[j] = x
    x_ref[...] = x


def kernel(inputs, A_stacked, B_stacked):
    L, B = inputs.shape
    N = A_stacked.shape[-1]
    h = N // 2
    inputs_t = inputs.reshape(L // _T, _T, B).transpose(0, 2, 1)

    grid = (L // _T,)
    return pl.pallas_call(
        _scan_body,
        out_shape=jax.ShapeDtypeStruct((L, B, N), jnp.float32),
        grid=grid,
        in_specs=[
            pl.BlockSpec((_T, h, h), lambda k: (k, 0, 0)),
            pl.BlockSpec((_T, h, N), lambda k: (k, 1, 0)),
            pl.BlockSpec((1, B, _T), lambda k: (k, 0, 0)),
            pl.BlockSpec((_T, N), lambda k: (k, 0)),
        ],
        out_specs=pl.BlockSpec((_T, B, N), lambda k: (k, 0, 0)),
        scratch_shapes=[pltpu.VMEM((B, N), jnp.float32)],
        compiler_params=pltpu.CompilerParams(
            dimension_semantics=("arbitrary",),
            vmem_limit_bytes=60 * 1024 * 1024,
        ),
        name="hippo_legs_scan",
    )(A_stacked, A_stacked, inputs_t, B_stacked)
